# baseline (speedup 1.0000x reference)
import jax
import jax.numpy as jnp
from jax import lax
from jax.experimental import pallas as pl
from jax.experimental.pallas import tpu as pltpu
from jax.experimental.pallas import tpu_sc as plsc

N_ASPECTS = 16
A_MIN, A_MAX = 1.0, 5.0
R_MIN, R_MAX = 1.0, 5.0
SCALE = (R_MAX - R_MIN) / (A_MAX - A_MIN)
OFFSET = R_MIN - SCALE * A_MIN

NUM_WORKERS = 32
CHUNK = 128


def _absa_body(uids, iids, ar, up_tab, ip_tab, out_u, out_i,
               uidx_v, iidx_v, urows_v, irows_v, ar_v, outu_v, outi_v, sem):
    b_per_w = outu_v.shape[0]
    n_chunks = b_per_w // CHUNK
    wid = lax.axis_index("s") * 2 + lax.axis_index("c")
    base = pl.multiple_of(wid * b_per_w, b_per_w)

    for j in range(n_chunks):
        pltpu.sync_copy(uids.at[pl.ds(base + j * CHUNK, CHUNK)], uidx_v.at[j])
        pltpu.sync_copy(iids.at[pl.ds(base + j * CHUNK, CHUNK)], iidx_v.at[j])

    copies = []
    for j in range(n_chunks):
        copies.append(pltpu.async_copy(
            up_tab.at[uidx_v.at[j]], urows_v.at[pl.ds(j * CHUNK, CHUNK)], sem))
        copies.append(pltpu.async_copy(
            ip_tab.at[iidx_v.at[j]], irows_v.at[pl.ds(j * CHUNK, CHUNK)], sem))
    pltpu.sync_copy(ar.at[pl.ds(base, b_per_w)], ar_v)
    for c in copies:
        c.wait()

    lanes = lax.iota(jnp.int32, 16)

    def group(g, carry):
        acc_u = jnp.zeros((16,), jnp.float32)
        acc_i = jnp.zeros((16,), jnp.float32)
        for j in range(16):
            e = g * 16 + j
            up = urows_v[e, :]
            ip = irows_v[e, :]
            arv = ar_v[e, :]
            upc = jnp.minimum(jnp.maximum(up, A_MIN), A_MAX)
            ipc = jnp.minimum(jnp.maximum(ip, A_MIN), A_MAX)
            su = jnp.sum(upc * arv, axis=0)
            si = jnp.sum(ipc * upc, axis=0)
            acc_u = jnp.where(lanes == j, su, acc_u)
            acc_i = jnp.where(lanes == j, si, acc_i)
        off = pl.multiple_of(g * 16, 16)
        outu_v[pl.ds(off, 16)] = acc_u * SCALE + OFFSET
        outi_v[pl.ds(off, 16)] = acc_i * SCALE + OFFSET
        return carry

    lax.fori_loop(0, b_per_w // 16, group, 0)

    pltpu.sync_copy(outu_v, out_u.at[pl.ds(base, b_per_w)])
    pltpu.sync_copy(outi_v, out_i.at[pl.ds(base, b_per_w)])


def kernel(U_ids, I_ids, A_ratings, users_parameters, items_parameters):
    B = U_ids.shape[0]
    b_per_w = B // NUM_WORKERS
    n_chunks = b_per_w // CHUNK
    mesh = plsc.VectorSubcoreMesh(core_axis_name="c", subcore_axis_name="s")
    f = pl.kernel(
        _absa_body,
        out_type=(
            jax.ShapeDtypeStruct((B,), jnp.float32),
            jax.ShapeDtypeStruct((B,), jnp.float32),
        ),
        mesh=mesh,
        compiler_params=pltpu.CompilerParams(
            needs_layout_passes=False, use_tc_tiling_on_sc=False),
        scratch_types=[
            pltpu.VMEM((n_chunks, CHUNK), jnp.int32),
            pltpu.VMEM((n_chunks, CHUNK), jnp.int32),
            pltpu.VMEM((b_per_w, N_ASPECTS), jnp.float32),
            pltpu.VMEM((b_per_w, N_ASPECTS), jnp.float32),
            pltpu.VMEM((b_per_w, N_ASPECTS), jnp.float32),
            pltpu.VMEM((b_per_w,), jnp.float32),
            pltpu.VMEM((b_per_w,), jnp.float32),
            pltpu.SemaphoreType.DMA,
        ],
    )
    return f(U_ids.astype(jnp.int32), I_ids.astype(jnp.int32),
             A_ratings, users_parameters, items_parameters)


# speedup vs baseline: 5.1900x; 5.1900x over previous
import jax
import jax.numpy as jnp
from jax import lax
from jax.experimental import pallas as pl
from jax.experimental.pallas import tpu as pltpu
from jax.experimental.pallas import tpu_sc as plsc

N_ASPECTS = 16
A_MIN, A_MAX = 1.0, 5.0
R_MIN, R_MAX = 1.0, 5.0
SCALE = (R_MAX - R_MIN) / (A_MAX - A_MIN)
OFFSET = R_MIN - SCALE * A_MIN

NUM_WORKERS = 32
W = 128
GROUPS = 32
GW = 16 * W


def _absa_body(uids, iids, ar_t, ut, it, out_u, out_i,
               uid_v, iid_v, buf, upc_v, ar_v, outu_v, outi_v, sem):
    b_per_w = GROUPS * 16
    wid = lax.axis_index("s") * 2 + lax.axis_index("c")
    base = pl.multiple_of(wid * b_per_w, b_per_w)

    pltpu.sync_copy(uids.at[pl.ds(base, b_per_w)], uid_v)
    pltpu.sync_copy(iids.at[pl.ds(base, b_per_w)], iid_v)
    pltpu.sync_copy(ar_t.at[:, pl.ds(base, b_per_w)], ar_v)

    lanes = lax.iota(jnp.int32, 16)

    def make_pass(ids_ref, tab, is_user):
        def fire(g, slot):
            off = pl.multiple_of(g * 16, 16)
            so = pl.multiple_of(slot * GW, GW)
            idv = ids_ref[pl.ds(off, 16)]
            for j in range(16):
                u = idv[j]
                ub = pl.multiple_of((u >> 7) << 7, 128)
                pltpu.async_copy(
                    tab.at[:, pl.ds(ub, W)],
                    buf.at[:, pl.ds(so + j * W, W)], sem)

        def drain(slot):
            so = pl.multiple_of(slot * GW, GW)
            pltpu.make_async_copy(
                tab.at[:, pl.ds(0, GW)], buf.at[:, pl.ds(so, GW)], sem).wait()

        def group(g, carry):
            slot = g & 1
            drain(slot)

            @pl.when(g < GROUPS - 1)
            def _():
                fire(g + 1, 1 - slot)

            off = pl.multiple_of(g * 16, 16)
            so = pl.multiple_of(slot * GW, GW)
            idv = ids_ref[pl.ds(off, 16)]
            col = so + (idv & (W - 1)) + lanes * W
            acc = jnp.zeros((16,), jnp.float32)
            for a in range(N_ASPECTS):
                row = jnp.full((16,), a, jnp.int32)
                vals = plsc.load_gather(buf, [row, col])
                valc = jnp.minimum(jnp.maximum(vals, A_MIN), A_MAX)
                if is_user:
                    upc_v[a, pl.ds(off, 16)] = valc
                    other = ar_v[a, pl.ds(off, 16)]
                else:
                    other = upc_v[a, pl.ds(off, 16)]
                acc = acc + valc * other
            out_ref = outu_v if is_user else outi_v
            out_ref[pl.ds(off, 16)] = acc * SCALE + OFFSET
            return carry

        fire(0, 0)
        lax.fori_loop(0, GROUPS, group, 0)

    make_pass(uid_v, ut, True)
    make_pass(iid_v, it, False)

    pltpu.sync_copy(outu_v, out_u.at[pl.ds(base, b_per_w)])
    pltpu.sync_copy(outi_v, out_i.at[pl.ds(base, b_per_w)])


def kernel(U_ids, I_ids, A_ratings, users_parameters, items_parameters):
    B = U_ids.shape[0]
    b_per_w = B // NUM_WORKERS
    mesh = plsc.VectorSubcoreMesh(core_axis_name="c", subcore_axis_name="s")
    f = pl.kernel(
        _absa_body,
        out_type=(
            jax.ShapeDtypeStruct((B,), jnp.float32),
            jax.ShapeDtypeStruct((B,), jnp.float32),
        ),
        mesh=mesh,
        compiler_params=pltpu.CompilerParams(
            needs_layout_passes=False, use_tc_tiling_on_sc=True),
        scratch_types=[
            pltpu.VMEM((b_per_w,), jnp.int32),
            pltpu.VMEM((b_per_w,), jnp.int32),
            pltpu.VMEM((16, 2 * GW), jnp.float32),
            pltpu.VMEM((16, b_per_w), jnp.float32),
            pltpu.VMEM((16, b_per_w), jnp.float32),
            pltpu.VMEM((b_per_w,), jnp.float32),
            pltpu.VMEM((b_per_w,), jnp.float32),
            pltpu.SemaphoreType.DMA,
        ],
    )
    return f(U_ids.astype(jnp.int32), I_ids.astype(jnp.int32),
             A_ratings.T, users_parameters.T, items_parameters.T)


# speedup vs baseline: 7.0413x; 1.3567x over previous
import jax
import jax.numpy as jnp
from jax import lax
from jax.experimental import pallas as pl
from jax.experimental.pallas import tpu as pltpu
from jax.experimental.pallas import tpu_sc as plsc

N_ASPECTS = 16
A_MIN, A_MAX = 1.0, 5.0
R_MIN, R_MAX = 1.0, 5.0
SCALE = (R_MAX - R_MIN) / (A_MAX - A_MIN)
OFFSET = R_MIN - SCALE * A_MIN

NUM_WORKERS = 32
W = 128
GROUPS = 32
GW = 16 * W


def _absa_body(uids, iids, ar_t, ut, it, out_u, out_i,
               uid_v, iid_v, buf, upc_v, ar_v, outu_v, outi_v, sem):
    b_per_w = GROUPS * 16
    wid = lax.axis_index("s") * 2 + lax.axis_index("c")
    base = pl.multiple_of(wid * b_per_w, b_per_w)

    pltpu.sync_copy(uids.at[pl.ds(base, b_per_w)], uid_v)
    pltpu.sync_copy(iids.at[pl.ds(base, b_per_w)], iid_v)
    pltpu.sync_copy(ar_t.at[:, pl.ds(base, b_per_w)], ar_v)

    lanes = lax.iota(jnp.int32, 16)

    def make_pass(ids_ref, tab, is_user):
        def fire(g, slot):
            off = pl.multiple_of(g * 16, 16)
            so = pl.multiple_of(slot * GW, GW)
            idv = ids_ref[pl.ds(off, 16)]
            for j in range(16):
                u = idv[j]
                ub = pl.multiple_of((u >> 7) << 7, 128)
                pltpu.async_copy(
                    tab.at[:, pl.ds(ub, W)],
                    buf.at[:, pl.ds(so + j * W, W)], sem)

        def drain(slot):
            so = pl.multiple_of(slot * GW, GW)
            pltpu.make_async_copy(
                tab.at[:, pl.ds(0, GW)], buf.at[:, pl.ds(so, GW)], sem).wait()

        def group(g, carry):
            slot = g - (g // 3) * 3
            drain(slot)

            nxt = g + 2
            nslot = nxt - (nxt // 3) * 3

            @pl.when(nxt < GROUPS)
            def _():
                fire(nxt, nslot)

            off = pl.multiple_of(g * 16, 16)
            so = pl.multiple_of(slot * GW, GW)
            idv = ids_ref[pl.ds(off, 16)]
            col = so + (idv & (W - 1)) + lanes * W
            acc = jnp.zeros((16,), jnp.float32)
            for a in range(N_ASPECTS):
                row = jnp.full((16,), a, jnp.int32)
                vals = plsc.load_gather(buf, [row, col])
                valc = jnp.minimum(jnp.maximum(vals, A_MIN), A_MAX)
                if is_user:
                    upc_v[a, pl.ds(off, 16)] = valc
                    other = ar_v[a, pl.ds(off, 16)]
                else:
                    other = upc_v[a, pl.ds(off, 16)]
                acc = acc + valc * other
            out_ref = outu_v if is_user else outi_v
            out_ref[pl.ds(off, 16)] = acc * SCALE + OFFSET
            return carry

        fire(0, 0)
        fire(1, 1)
        lax.fori_loop(0, GROUPS, group, 0)

    make_pass(uid_v, ut, True)
    make_pass(iid_v, it, False)

    pltpu.sync_copy(outu_v, out_u.at[pl.ds(base, b_per_w)])
    pltpu.sync_copy(outi_v, out_i.at[pl.ds(base, b_per_w)])


def kernel(U_ids, I_ids, A_ratings, users_parameters, items_parameters):
    B = U_ids.shape[0]
    b_per_w = B // NUM_WORKERS
    mesh = plsc.VectorSubcoreMesh(core_axis_name="c", subcore_axis_name="s")
    f = pl.kernel(
        _absa_body,
        out_type=(
            jax.ShapeDtypeStruct((B,), jnp.float32),
            jax.ShapeDtypeStruct((B,), jnp.float32),
        ),
        mesh=mesh,
        compiler_params=pltpu.CompilerParams(
            needs_layout_passes=False, use_tc_tiling_on_sc=True),
        scratch_types=[
            pltpu.VMEM((b_per_w,), jnp.int32),
            pltpu.VMEM((b_per_w,), jnp.int32),
            pltpu.VMEM((16, 3 * GW), jnp.float32),
            pltpu.VMEM((16, b_per_w), jnp.float32),
            pltpu.VMEM((16, b_per_w), jnp.float32),
            pltpu.VMEM((b_per_w,), jnp.float32),
            pltpu.VMEM((b_per_w,), jnp.float32),
            pltpu.SemaphoreType.DMA,
        ],
    )
    return f(U_ids.astype(jnp.int32), I_ids.astype(jnp.int32),
             A_ratings.T, users_parameters.T, items_parameters.T)


# speedup vs baseline: 7.1140x; 1.0103x over previous
import jax
import jax.numpy as jnp
from jax import lax
from jax.experimental import pallas as pl
from jax.experimental.pallas import tpu as pltpu
from jax.experimental.pallas import tpu_sc as plsc

N_ASPECTS = 16
A_MIN, A_MAX = 1.0, 5.0
R_MIN, R_MAX = 1.0, 5.0
SCALE = (R_MAX - R_MIN) / (A_MAX - A_MIN)
OFFSET = R_MIN - SCALE * A_MIN

NUM_WORKERS = 32
W = 128
GROUPS = 32
GW = 16 * W


def _absa_body(uids, iids, ar_t, ut, it, out_u, out_i,
               uid_v, iid_v, buf, upc_v, ar_v, outu_v, outi_v, sem):
    b_per_w = GROUPS * 16
    wid = lax.axis_index("s") * 2 + lax.axis_index("c")
    base = pl.multiple_of(wid * b_per_w, b_per_w)

    pltpu.sync_copy(uids.at[pl.ds(base, b_per_w)], uid_v)
    pltpu.sync_copy(iids.at[pl.ds(base, b_per_w)], iid_v)
    pltpu.sync_copy(ar_t.at[:, pl.ds(base, b_per_w)], ar_v)

    lanes = lax.iota(jnp.int32, 16)

    def make_pass(ids_ref, tab, is_user):
        def fire(g, slot):
            off = pl.multiple_of(g * 16, 16)
            so = pl.multiple_of(slot * GW, GW)
            idv = ids_ref[pl.ds(off, 16)]
            for j in range(16):
                u = idv[j]
                ub = pl.multiple_of((u >> 7) << 7, 128)
                pltpu.async_copy(
                    tab.at[pl.ds(0, 8), pl.ds(ub, W)],
                    buf.at[pl.ds(0, 8), pl.ds(so + j * W, W)], sem)
                pltpu.async_copy(
                    tab.at[pl.ds(8, 8), pl.ds(ub, W)],
                    buf.at[pl.ds(8, 8), pl.ds(so + j * W, W)], sem)

        def drain(slot):
            so = pl.multiple_of(slot * GW, GW)
            pltpu.make_async_copy(
                tab.at[:, pl.ds(0, GW)], buf.at[:, pl.ds(so, GW)], sem).wait()

        def group(g, carry):
            slot = g - (g // 3) * 3
            drain(slot)

            nxt = g + 2
            nslot = nxt - (nxt // 3) * 3

            @pl.when(nxt < GROUPS)
            def _():
                fire(nxt, nslot)

            off = pl.multiple_of(g * 16, 16)
            so = pl.multiple_of(slot * GW, GW)
            idv = ids_ref[pl.ds(off, 16)]
            col = so + (idv & (W - 1)) + lanes * W
            acc = jnp.zeros((16,), jnp.float32)
            for a in range(N_ASPECTS):
                row = jnp.full((16,), a, jnp.int32)
                vals = plsc.load_gather(buf, [row, col])
                valc = jnp.minimum(jnp.maximum(vals, A_MIN), A_MAX)
                if is_user:
                    upc_v[a, pl.ds(off, 16)] = valc
                    other = ar_v[a, pl.ds(off, 16)]
                else:
                    other = upc_v[a, pl.ds(off, 16)]
                acc = acc + valc * other
            out_ref = outu_v if is_user else outi_v
            out_ref[pl.ds(off, 16)] = acc * SCALE + OFFSET
            return carry

        fire(0, 0)
        fire(1, 1)
        lax.fori_loop(0, GROUPS, group, 0)

    make_pass(uid_v, ut, True)
    make_pass(iid_v, it, False)

    pltpu.sync_copy(outu_v, out_u.at[pl.ds(base, b_per_w)])
    pltpu.sync_copy(outi_v, out_i.at[pl.ds(base, b_per_w)])


def kernel(U_ids, I_ids, A_ratings, users_parameters, items_parameters):
    B = U_ids.shape[0]
    b_per_w = B // NUM_WORKERS
    mesh = plsc.VectorSubcoreMesh(core_axis_name="c", subcore_axis_name="s")
    f = pl.kernel(
        _absa_body,
        out_type=(
            jax.ShapeDtypeStruct((B,), jnp.float32),
            jax.ShapeDtypeStruct((B,), jnp.float32),
        ),
        mesh=mesh,
        compiler_params=pltpu.CompilerParams(
            needs_layout_passes=False, use_tc_tiling_on_sc=True),
        scratch_types=[
            pltpu.VMEM((b_per_w,), jnp.int32),
            pltpu.VMEM((b_per_w,), jnp.int32),
            pltpu.VMEM((16, 3 * GW), jnp.float32),
            pltpu.VMEM((16, b_per_w), jnp.float32),
            pltpu.VMEM((16, b_per_w), jnp.float32),
            pltpu.VMEM((b_per_w,), jnp.float32),
            pltpu.VMEM((b_per_w,), jnp.float32),
            pltpu.SemaphoreType.DMA,
        ],
    )
    return f(U_ids.astype(jnp.int32), I_ids.astype(jnp.int32),
             A_ratings.T, users_parameters.T, items_parameters.T)
